# baseline (speedup 1.0000x reference)
import functools

import jax
import jax.numpy as jnp
from jax import lax
from jax.experimental import pallas as pl
from jax.experimental.pallas import tpu as pltpu
from jax.experimental.pallas import tpu_sc as plsc

_NUM_FIELDS = 26
_VOCAB = 100000
_EMB_DIM = 32
_BATCH = 16384

_NW = 32
_OUT_CH = 4096
_NCH = _BATCH // _OUT_CH
_VPC = _OUT_CH // 128

_mesh = plsc.VectorSubcoreMesh(core_axis_name="c", subcore_axis_name="s")


@functools.partial(
    pl.kernel,
    mesh=_mesh,
    compiler_params=pltpu.CompilerParams(
        use_tc_tiling_on_sc=True, needs_layout_passes=False
    ),
    out_type=jax.ShapeDtypeStruct((_NUM_FIELDS * _EMB_DIM, _BATCH), jnp.float32),
    scratch_types=[
        pltpu.VMEM((_VOCAB,), jnp.float32),
        pltpu.VMEM((_BATCH,), jnp.int32),
        pltpu.VMEM((2, _OUT_CH), jnp.float32),
        pltpu.SemaphoreType.DMA,
        pltpu.SemaphoreType.DMA,
    ],
)
def _lookup_all(tt_hbm, xt_hbm, out_hbm, src_v, idx_v, ob_v, sem0, sem1):
    wid = lax.axis_index("s") * 2 + lax.axis_index("c")
    sems = (sem0, sem1)

    def field_body(f, _):
        pltpu.sync_copy(xt_hbm.at[f, :], idx_v)
        pltpu.sync_copy(tt_hbm.at[f, wid, :], src_v)
        c = f * _EMB_DIM + wid
        pend = [None, None]
        for h in range(_NCH):
            p = h % 2
            if pend[p] is not None:
                pend[p].wait()

            @plsc.parallel_loop(0, _OUT_CH // 16, unroll=16)
            def chunk_body(i, h=h, p=p):
                o = h * _OUT_CH + i * 16
                iv = idx_v[pl.ds(o, 16)]
                ob_v[p, pl.ds(i * 16, 16)] = plsc.load_gather(src_v, [iv])

            pend[p] = pltpu.async_copy(
                ob_v.at[p], out_hbm.at[c, pl.ds(h * _OUT_CH, _OUT_CH)], sems[p]
            )
        pend[0].wait()
        pend[1].wait()
        return 0

    lax.fori_loop(0, _NUM_FIELDS, field_body, 0)


def kernel(x, tables):
    tt = tables.transpose(0, 2, 1)
    xt = x.T
    out_t = _lookup_all(tt, xt)
    return out_t.T
